# baseline (speedup 1.0000x reference)
import jax
import jax.numpy as jnp
import numpy as np
from jax.experimental import pallas as pl
from jax.experimental.pallas import tpu as pltpu
from jax.sharding import Mesh, PartitionSpec as P

_CONTRACT_LAST = (((1,), (1,)), ((), ()))


def _mm_cast_oneshot_kernel(x_ref, w_ref, o_ref):
    o_ref[...] = jax.lax.dot_general(
        x_ref[...].astype(jnp.bfloat16), w_ref[...].astype(jnp.bfloat16),
        dimension_numbers=_CONTRACT_LAST,
        preferred_element_type=jnp.float32,
    )


def _linear_oneshot(x2d, w, tm, tn):
    M, K = x2d.shape
    N = w.shape[0]
    grid = (N // tn, M // tm)
    out = pl.pallas_call(
        _mm_cast_oneshot_kernel,
        out_shape=jax.ShapeDtypeStruct((M, N), jnp.float32),
        grid=grid,
        in_specs=[
            pl.BlockSpec((tm, K), lambda j, i: (i, 0)),
            pl.BlockSpec((tn, K), lambda j, i: (j, 0)),
        ],
        out_specs=pl.BlockSpec((tm, tn), lambda j, i: (i, j)),
        compiler_params=pltpu.CompilerParams(
            dimension_semantics=("parallel", "arbitrary"),
            vmem_limit_bytes=60 << 20,
        ),
        cost_estimate=pl.CostEstimate(
            flops=2 * M * N * K,
            bytes_accessed=(M * K * (N // tn) + N * K) * 4 + M * N * 4,
            transcendentals=0,
        ),
    )(x2d, w)
    return out


def _mm_cast_accum_kernel(x_ref, w_ref, o_ref):
    acc = jax.lax.dot_general(
        x_ref[...], w_ref[...],
        dimension_numbers=_CONTRACT_LAST,
        preferred_element_type=jnp.float32,
    )

    @pl.when(pl.program_id(2) == 0)
    def _():
        o_ref[...] = acc

    @pl.when(pl.program_id(2) != 0)
    def _():
        o_ref[...] += acc


def _linear_fused(x2d, w, tm, tn, tk):
    M, K = x2d.shape
    N = w.shape[0]
    grid = (N // tn, M // tm, K // tk)
    out = pl.pallas_call(
        _mm_cast_accum_kernel,
        out_shape=jax.ShapeDtypeStruct((M, N), jnp.float32),
        grid=grid,
        in_specs=[
            pl.BlockSpec((tm, tk), lambda j, i, k: (i, k)),
            pl.BlockSpec((tn, tk), lambda j, i, k: (j, k)),
        ],
        out_specs=pl.BlockSpec((tm, tn), lambda j, i, k: (i, j)),
        compiler_params=pltpu.CompilerParams(
            dimension_semantics=("parallel", "parallel", "arbitrary"),
            vmem_limit_bytes=64 << 20,
        ),
        cost_estimate=pl.CostEstimate(
            flops=2 * M * N * K,
            bytes_accessed=(M * K + N * K) * 4 + M * N * 4,
            transcendentals=0,
        ),
    )(x2d, w)
    return out


def kernel(x, weight):
    orig_lead = x.shape[:-1]
    K = x.shape[-1]
    N = weight.shape[0]
    x2d = x.reshape(-1, K)
    M = x2d.shape[0]

    out = _linear_fused(x2d, weight, tm=2048, tn=2048, tk=512)
    return out.reshape(*orig_lead, N)


# speedup vs baseline: 1.2089x; 1.2089x over previous
import jax
import jax.numpy as jnp
import numpy as np
from jax.experimental import pallas as pl
from jax.experimental.pallas import tpu as pltpu
from jax.sharding import Mesh, PartitionSpec as P

_CONTRACT_LAST = (((1,), (1,)), ((), ()))


def _mm_cast_oneshot_kernel(x_ref, w_ref, o_ref):
    o_ref[...] = jax.lax.dot_general(
        x_ref[...].astype(jnp.bfloat16), w_ref[...].astype(jnp.bfloat16),
        dimension_numbers=_CONTRACT_LAST,
        preferred_element_type=jnp.float32,
    )


def _linear_oneshot(x2d, w, tm, tn):
    M, K = x2d.shape
    N = w.shape[0]
    grid = (N // tn, M // tm)
    out = pl.pallas_call(
        _mm_cast_oneshot_kernel,
        out_shape=jax.ShapeDtypeStruct((M, N), jnp.float32),
        grid=grid,
        in_specs=[
            pl.BlockSpec((tm, K), lambda j, i: (i, 0)),
            pl.BlockSpec((tn, K), lambda j, i: (j, 0)),
        ],
        out_specs=pl.BlockSpec((tm, tn), lambda j, i: (i, j)),
        compiler_params=pltpu.CompilerParams(
            dimension_semantics=("parallel", "arbitrary"),
            vmem_limit_bytes=60 << 20,
        ),
        cost_estimate=pl.CostEstimate(
            flops=2 * M * N * K,
            bytes_accessed=(M * K * (N // tn) + N * K) * 4 + M * N * 4,
            transcendentals=0,
        ),
    )(x2d, w)
    return out


def _mm_cast_accum_kernel(x_ref, w_ref, o_ref):
    prev = jnp.where(pl.program_id(2) == 0, jnp.zeros_like(o_ref), o_ref[...])
    o_ref[...] = prev + jax.lax.dot_general(
        x_ref[...], w_ref[...],
        dimension_numbers=_CONTRACT_LAST,
        preferred_element_type=jnp.float32,
    )


def _linear_fused(x2d, w, tm, tn, tk):
    M, K = x2d.shape
    N = w.shape[0]
    grid = (N // tn, M // tm, K // tk)
    out = pl.pallas_call(
        _mm_cast_accum_kernel,
        out_shape=jax.ShapeDtypeStruct((M, N), jnp.float32),
        grid=grid,
        in_specs=[
            pl.BlockSpec((tm, tk), lambda j, i, k: (i, k)),
            pl.BlockSpec((tn, tk), lambda j, i, k: (j, k)),
        ],
        out_specs=pl.BlockSpec((tm, tn), lambda j, i, k: (i, j)),
        compiler_params=pltpu.CompilerParams(
            dimension_semantics=("parallel", "parallel", "arbitrary"),
            vmem_limit_bytes=60 << 20,
        ),
        cost_estimate=pl.CostEstimate(
            flops=2 * M * N * K,
            bytes_accessed=(M * K + N * K) * 4 + M * N * 4,
            transcendentals=0,
        ),
    )(x2d, w)
    return out


def kernel(x, weight):
    orig_lead = x.shape[:-1]
    K = x.shape[-1]
    N = weight.shape[0]
    x2d = x.reshape(-1, K)
    M = x2d.shape[0]

    out = _linear_fused(x2d, weight, tm=2048, tn=2048, tk=512)
    return out.reshape(*orig_lead, N)


# speedup vs baseline: 1.2512x; 1.0350x over previous
import jax
import jax.numpy as jnp
from jax.experimental import pallas as pl
from jax.experimental.pallas import tpu as pltpu

_CONTRACT_LAST = (((1,), (1,)), ((), ()))


def _mm_accum_kernel(x_ref, w_ref, o_ref):
    @pl.when(pl.program_id(2) == 0)
    def _():
        o_ref[...] = jnp.zeros_like(o_ref)

    o_ref[...] += jax.lax.dot_general(
        x_ref[...], w_ref[...],
        dimension_numbers=_CONTRACT_LAST,
        preferred_element_type=jnp.float32,
    )


def _linear_fused(x2d, w, tm, tn, tk):
    M, K = x2d.shape
    N = w.shape[0]
    grid = (N // tn, M // tm, K // tk)
    out = pl.pallas_call(
        _mm_accum_kernel,
        out_shape=jax.ShapeDtypeStruct((M, N), jnp.float32),
        grid=grid,
        in_specs=[
            pl.BlockSpec((tm, tk), lambda j, i, k: (i, k)),
            pl.BlockSpec((tn, tk), lambda j, i, k: (j, k)),
        ],
        out_specs=pl.BlockSpec((tm, tn), lambda j, i, k: (i, j)),
        compiler_params=pltpu.CompilerParams(
            dimension_semantics=("parallel", "parallel", "arbitrary"),
            vmem_limit_bytes=60 << 20,
        ),
        cost_estimate=pl.CostEstimate(
            flops=2 * M * N * K,
            bytes_accessed=(M * K + N * K) * 4 + M * N * 4,
            transcendentals=0,
        ),
    )(x2d, w)
    return out


def kernel(x, weight):
    orig_lead = x.shape[:-1]
    K = x.shape[-1]
    N = weight.shape[0]
    x2d = x.reshape(-1, K)
    out = _linear_fused(x2d, weight, tm=2048, tn=2048, tk=512)
    return out.reshape(*orig_lead, N)
